# baseline (speedup 1.0000x reference)
import functools

import jax
import jax.numpy as jnp
from jax import lax
from jax.experimental import pallas as pl
from jax.experimental.pallas import tpu as pltpu
from jax.experimental.pallas import tpu_sc as plsc

R, N = 64, 8192
L = 16
NV = N // L
_INFO = plsc.get_sparse_core_info()
NC, NS = _INFO.num_cores, _INFO.num_subcores
NW = NC * NS
RPW = R // NW
B_MAX = 26
BUF = N + L
UNROLL = 8
CH = 2
NCH = N // CH

_mesh = plsc.VectorSubcoreMesh(core_axis_name="c", subcore_axis_name="s")


@functools.partial(
    pl.kernel,
    out_type=jax.ShapeDtypeStruct((R, N), jnp.float32),
    mesh=_mesh,
    compiler_params=pltpu.CompilerParams(needs_layout_passes=False),
    scratch_types=[
        pltpu.VMEM((RPW * N,), jnp.float32),
        pltpu.VMEM((RPW * BUF,), jnp.float32),
        pltpu.VMEM((RPW * N,), jnp.float32),
    ] + [pltpu.SemaphoreType.DMA] * (RPW * CH)
      + [pltpu.SemaphoreType.DMA] * (RPW * 4),
)
def _sparsemax_sc(x_hbm, out_hbm, x_v, buf_v, y_v, *sems):
    in_sems, out_sems = sems[:RPW * CH], sems[RPW * CH:]
    wid = lax.axis_index("s") * NC + lax.axis_index("c")
    base = wid * RPW
    in_copies = []
    for c in range(CH):
        for r in range(RPW):
            in_copies.append(pltpu.async_copy(
                x_hbm.at[base + r, pl.ds(c * NCH, NCH)],
                x_v.at[pl.ds(r * N + c * NCH, NCH)],
                in_sems[c * RPW + r]))

    def max_body(i, accs):
        b = i * (UNROLL * L)
        out = []
        for r in range(RPW):
            vs = [x_v[pl.ds(r * N + b + u * L, L)] for u in range(UNROLL)]
            while len(vs) > 1:
                vs = [jnp.maximum(vs[j], vs[j + 1]) for j in range(0, len(vs), 2)]
            out.append(jnp.maximum(accs[r], vs[0]))
        return tuple(out)

    accs = (jnp.full((L,), -jnp.inf, jnp.float32),) * RPW
    per_ch = NCH // (UNROLL * L)
    for c in range(CH):
        for r in range(RPW):
            in_copies[c * RPW + r].wait()
        accs = lax.fori_loop(c * per_ch, (c + 1) * per_ch, max_body, accs)
    m = [jnp.max(a) for a in accs]
    thr = [mm - 1.0 for mm in m]

    CUNROLL = 8

    def comp_body(i, offs):
        b = i * (CUNROLL * L)
        vals = [[x_v[pl.ds(r * N + b + u * L, L)] for u in range(CUNROLL)]
                for r in range(RPW)]
        msks = [[vals[r][u] > thr[r] for u in range(CUNROLL)]
                for r in range(RPW)]
        pcs = [[plsc.all_reduce_population_count(msks[r][u])[0]
                for u in range(CUNROLL)] for r in range(RPW)]
        offs = list(offs)
        for u in range(CUNROLL):
            for r in range(RPW):
                plsc.store_compressed(
                    buf_v.at[pl.ds(r * BUF + offs[r], L)], vals[r][u],
                    mask=msks[r][u])
                offs[r] = offs[r] + pcs[r][u]
        return tuple(offs)

    cnts = lax.fori_loop(0, NV // CUNROLL, comp_body, (jnp.int32(0),) * RPW)
    nv = [(c + (L - 1)) >> 4 for c in cnts]
    nvm = nv[0]
    for r in range(1, RPW):
        nvm = jnp.maximum(nvm, nv[r])
    nvl = nvm * L

    for r in range(RPW):
        pad = jnp.full((L,), thr[r] - 1.0, jnp.float32)

        def pad_body(off, r=r, pad=pad):
            buf_v[pl.ds(r * BUF + off, L)] = pad
            return off + L

        lax.while_loop(lambda off: off < nvl, pad_body, cnts[r])

    cnt_f = [cnts[r].astype(jnp.float32) for r in range(RPW)]

    def bis_cond(carry):
        it, lo, hi = carry
        wide = (hi[0] - lo[0]) * cnt_f[0] > 0.005
        for r in range(1, RPW):
            wide = wide | ((hi[r] - lo[r]) * cnt_f[r] > 0.005)
        return wide & (it < B_MAX)

    def bis_body(carry):
        it, lo, hi = carry
        tau = [0.5 * (lo[r] + hi[r]) for r in range(RPW)]

        def g_body(i, accs):
            return tuple(
                accs[r] + jnp.maximum(buf_v[pl.ds(r * BUF + i * L, L)]
                                      - tau[r], 0.0)
                for r in range(RPW))

        z = jnp.zeros((L,), jnp.float32)
        accs = lax.fori_loop(0, nvm, g_body, (z,) * RPW)
        ok = [(jnp.sum(accs[r]) - 1.0) >= 0.0 for r in range(RPW)]
        return (it + 1,
                tuple(jnp.where(ok[r], tau[r], lo[r]) for r in range(RPW)),
                tuple(jnp.where(ok[r], hi[r], tau[r]) for r in range(RPW)))

    _, lo, _ = lax.while_loop(bis_cond, bis_body,
                              (jnp.int32(0), tuple(thr), tuple(m)))

    def cs_body(i, carry):
        c, s = carry
        c, s = list(c), list(s)
        for r in range(RPW):
            v = buf_v[pl.ds(r * BUF + i * L, L)]
            msk = v > lo[r]
            c[r] = c[r] + jnp.where(msk, 1.0, 0.0)
            s[r] = s[r] + jnp.where(msk, v, 0.0)
        return tuple(c), tuple(s)

    z = jnp.zeros((L,), jnp.float32)
    c, s = lax.fori_loop(0, nvm, cs_body, ((z,) * RPW, (z,) * RPW))
    tau = [jnp.broadcast_to(jnp.sum(s[r]) - 1.0, (L,)) /
           jnp.broadcast_to(jnp.sum(c[r]), (L,)) for r in range(RPW)]

    def out_body(i, carry):
        b = i * (UNROLL * L)
        ys = [[jnp.maximum(x_v[pl.ds(r * N + b + u * L, L)] - tau[r], 0.0)
               for u in range(UNROLL)] for r in range(RPW)]
        for u in range(UNROLL):
            for r in range(RPW):
                y_v[pl.ds(r * N + b + u * L, L)] = ys[r][u]
        return carry

    N4 = N // 4
    out_copies = []
    for h in range(4):
        lax.fori_loop(h * (NV // UNROLL // 4), (h + 1) * (NV // UNROLL // 4),
                      out_body, 0)
        for r in range(RPW):
            out_copies.append(pltpu.async_copy(
                y_v.at[pl.ds(r * N + h * N4, N4)],
                out_hbm.at[base + r, pl.ds(h * N4, N4)],
                out_sems[h * RPW + r]))
    for cp in out_copies:
        cp.wait()


def kernel(input):
    return _sparsemax_sc(input)


# speedup vs baseline: 1.0169x; 1.0169x over previous
import functools

import jax
import jax.numpy as jnp
from jax import lax
from jax.experimental import pallas as pl
from jax.experimental.pallas import tpu as pltpu
from jax.experimental.pallas import tpu_sc as plsc

R, N = 64, 8192
L = 16
NV = N // L
_INFO = plsc.get_sparse_core_info()
NC, NS = _INFO.num_cores, _INFO.num_subcores
NW = NC * NS
RPW = R // NW
B_MAX = 26
BUF = N + L
UNROLL = 8
CH = 2
NCH = N // CH

_mesh = plsc.VectorSubcoreMesh(core_axis_name="c", subcore_axis_name="s")


@functools.partial(
    pl.kernel,
    out_type=jax.ShapeDtypeStruct((R, N), jnp.float32),
    mesh=_mesh,
    compiler_params=pltpu.CompilerParams(needs_layout_passes=False),
    scratch_types=[
        pltpu.VMEM((RPW * N,), jnp.float32),
        pltpu.VMEM((RPW * BUF,), jnp.float32),
        pltpu.VMEM((RPW * N,), jnp.float32),
    ] + [pltpu.SemaphoreType.DMA] * (RPW * CH)
      + [pltpu.SemaphoreType.DMA] * (RPW * 2),
)
def _sparsemax_sc(x_hbm, out_hbm, x_v, buf_v, y_v, *sems):
    in_sems, out_sems = sems[:RPW * CH], sems[RPW * CH:]
    wid = lax.axis_index("s") * NC + lax.axis_index("c")
    base = wid * RPW
    in_copies = []
    for c in range(CH):
        for r in range(RPW):
            in_copies.append(pltpu.async_copy(
                x_hbm.at[base + r, pl.ds(c * NCH, NCH)],
                x_v.at[pl.ds(r * N + c * NCH, NCH)],
                in_sems[c * RPW + r]))

    def max_body(i, accs):
        b = i * (UNROLL * L)
        out = []
        for r in range(RPW):
            vs = [x_v[pl.ds(r * N + b + u * L, L)] for u in range(UNROLL)]
            while len(vs) > 1:
                vs = [jnp.maximum(vs[j], vs[j + 1]) for j in range(0, len(vs), 2)]
            out.append(jnp.maximum(accs[r], vs[0]))
        return tuple(out)

    accs = (jnp.full((L,), -jnp.inf, jnp.float32),) * RPW
    per_ch = NCH // (UNROLL * L)
    for c in range(CH):
        for r in range(RPW):
            in_copies[c * RPW + r].wait()
        accs = lax.fori_loop(c * per_ch, (c + 1) * per_ch, max_body, accs)
    m = [jnp.max(a) for a in accs]
    thr = [mm - 1.0 for mm in m]

    CUNROLL = 8

    def comp_body(i, offs):
        b = i * (CUNROLL * L)
        vals = [[x_v[pl.ds(r * N + b + u * L, L)] for u in range(CUNROLL)]
                for r in range(RPW)]
        msks = [[vals[r][u] > thr[r] for u in range(CUNROLL)]
                for r in range(RPW)]
        pcs = [[plsc.all_reduce_population_count(msks[r][u])[0]
                for u in range(CUNROLL)] for r in range(RPW)]
        offs = list(offs)
        for u in range(CUNROLL):
            for r in range(RPW):
                plsc.store_compressed(
                    buf_v.at[pl.ds(r * BUF + offs[r], L)], vals[r][u],
                    mask=msks[r][u])
                offs[r] = offs[r] + pcs[r][u]
        return tuple(offs)

    cnts = lax.fori_loop(0, NV // CUNROLL, comp_body, (jnp.int32(0),) * RPW)
    nv = [(c + (L - 1)) >> 4 for c in cnts]
    nvm = nv[0]
    for r in range(1, RPW):
        nvm = jnp.maximum(nvm, nv[r])
    lane = lax.iota(jnp.int32, L)

    cnt_f = [cnts[r].astype(jnp.float32) for r in range(RPW)]

    def bis_cond(carry):
        it, lo, hi = carry
        wide = (hi[0] - lo[0]) * cnt_f[0] > 0.005
        for r in range(1, RPW):
            wide = wide | ((hi[r] - lo[r]) * cnt_f[r] > 0.005)
        return wide & (it < B_MAX)

    def bis_body(carry):
        it, lo, hi = carry
        tau = [0.5 * (lo[r] + hi[r]) for r in range(RPW)]

        def g_body(i, accs):
            idx = lane + i * L
            out = []
            for r in range(RPW):
                v = buf_v[pl.ds(r * BUF + i * L, L)]
                rl = jnp.maximum(v - tau[r], 0.0)
                out.append(accs[r] + jnp.where(idx < cnts[r], rl, 0.0))
            return tuple(out)

        z = jnp.zeros((L,), jnp.float32)
        accs = lax.fori_loop(0, nvm, g_body, (z,) * RPW)
        ok = [(jnp.sum(accs[r]) - 1.0) >= 0.0 for r in range(RPW)]
        return (it + 1,
                tuple(jnp.where(ok[r], tau[r], lo[r]) for r in range(RPW)),
                tuple(jnp.where(ok[r], hi[r], tau[r]) for r in range(RPW)))

    _, lo, _ = lax.while_loop(bis_cond, bis_body,
                              (jnp.int32(0), tuple(thr), tuple(m)))

    def cs_body(i, carry):
        c, s = carry
        c, s = list(c), list(s)
        idx = lane + i * L
        for r in range(RPW):
            v = buf_v[pl.ds(r * BUF + i * L, L)]
            msk = (v > lo[r]) & (idx < cnts[r])
            c[r] = c[r] + jnp.where(msk, 1.0, 0.0)
            s[r] = s[r] + jnp.where(msk, v, 0.0)
        return tuple(c), tuple(s)

    z = jnp.zeros((L,), jnp.float32)
    c, s = lax.fori_loop(0, nvm, cs_body, ((z,) * RPW, (z,) * RPW))
    tau = [jnp.broadcast_to(jnp.sum(s[r]) - 1.0, (L,)) /
           jnp.broadcast_to(jnp.sum(c[r]), (L,)) for r in range(RPW)]

    def out_body(i, carry):
        b = i * (UNROLL * L)
        ys = [[jnp.maximum(x_v[pl.ds(r * N + b + u * L, L)] - tau[r], 0.0)
               for u in range(UNROLL)] for r in range(RPW)]
        for u in range(UNROLL):
            for r in range(RPW):
                y_v[pl.ds(r * N + b + u * L, L)] = ys[r][u]
        return carry

    N2 = N // 2
    out_copies = []
    for h in range(2):
        lax.fori_loop(h * (NV // UNROLL // 2), (h + 1) * (NV // UNROLL // 2),
                      out_body, 0)
        for r in range(RPW):
            out_copies.append(pltpu.async_copy(
                y_v.at[pl.ds(r * N + h * N2, N2)],
                out_hbm.at[base + r, pl.ds(h * N2, N2)],
                out_sems[h * RPW + r]))
    for cp in out_copies:
        cp.wait()


def kernel(input):
    return _sparsemax_sc(input)
